# Initial kernel scaffold; baseline (speedup 1.0000x reference)
import jax
import jax.numpy as jnp
from jax.experimental import pallas as pl


def kernel(x, pos, residue_number, batch, W1, b1, W2, b2):
    raise NotImplementedError("write your pallas kernel here")


# speedup vs baseline: 24.9575x; 24.9575x over previous
import functools

import jax
import jax.numpy as jnp
from jax.experimental import pallas as pl
from jax.experimental.pallas import tpu as pltpu

_N = 50000
_D = 128
_M = 12500
_H = 64
_R = 391
_NPAD = _R * 128


def _fps_body(px_ref, py_ref, pz_ref, ps_ref, idx_ref, dists_ref, iota_ref):
    iota = (jax.lax.broadcasted_iota(jnp.int32, (_R, 128), 0) * 128
            + jax.lax.broadcasted_iota(jnp.int32, (_R, 128), 1))
    iota_ref[...] = iota
    valid = iota < _N
    dists_ref[...] = jnp.where(valid, jnp.inf, -jnp.inf).astype(jnp.float32)
    idx_ref[0] = 0

    def body(i, last):
        sx = ps_ref[0, last]
        sy = ps_ref[1, last]
        sz = ps_ref[2, last]
        dx = px_ref[...] - sx
        dy = py_ref[...] - sy
        dz = pz_ref[...] - sz
        d = (dx * dx + dz * dz) + dy * dy
        dn = jnp.minimum(dists_ref[...], d)
        dists_ref[...] = dn
        m = jnp.max(dn)
        nxt = jnp.min(jnp.where(dn == m, iota_ref[...], jnp.int32(0x7FFFFFFF)))
        idx_ref[i] = nxt
        return nxt

    jax.lax.fori_loop(1, _M, body, jnp.int32(0))


def _fps(pos):
    px = jnp.zeros((_NPAD,), jnp.float32).at[:_N].set(pos[:, 0]).reshape(_R, 128)
    py = jnp.zeros((_NPAD,), jnp.float32).at[:_N].set(pos[:, 1]).reshape(_R, 128)
    pz = jnp.zeros((_NPAD,), jnp.float32).at[:_N].set(pos[:, 2]).reshape(_R, 128)
    return pl.pallas_call(
        _fps_body,
        out_shape=jax.ShapeDtypeStruct((_M,), jnp.int32),
        in_specs=[
            pl.BlockSpec(memory_space=pltpu.VMEM),
            pl.BlockSpec(memory_space=pltpu.VMEM),
            pl.BlockSpec(memory_space=pltpu.VMEM),
            pl.BlockSpec(memory_space=pltpu.SMEM),
        ],
        out_specs=pl.BlockSpec(memory_space=pltpu.SMEM),
        scratch_shapes=[
            pltpu.VMEM((_R, 128), jnp.float32),
            pltpu.VMEM((_R, 128), jnp.int32),
        ],
    )(px, py, pz, pos.T)


def kernel(x, pos, residue_number, batch, W1, b1, W2, b2):
    idx = _fps(pos)
    row = residue_number
    pos_dst = jnp.take(pos, idx, axis=0)
    rel = pos - jnp.take(pos_dst, row, axis=0)
    msg = jnp.concatenate([x, rel], axis=-1)
    h = jnp.maximum(msg @ W1 + b1, 0.0)
    h = h @ W2 + b2
    out = jax.ops.segment_max(h, row, num_segments=_M)
    out = jnp.where(jnp.isfinite(out), out, 0.0)
    batch_new = jnp.take(batch, idx, axis=0)
    return out, pos_dst, batch_new


# speedup vs baseline: 25.7496x; 1.0317x over previous
import dataclasses
import functools

import jax
import jax.numpy as jnp
from jax import lax
from jax.experimental import pallas as pl
from jax.experimental.pallas import tpu as pltpu
from jax.experimental.pallas import tpu_sc as plsc

_N = 50000
_D = 128
_M = 12500
_H = 64
_R = 391
_NPAD = _R * 128


def _fps_body(px_ref, py_ref, pz_ref, ps_ref, idx_ref, dists_ref, iota_ref):
    iota = (jax.lax.broadcasted_iota(jnp.int32, (_R, 128), 0) * 128
            + jax.lax.broadcasted_iota(jnp.int32, (_R, 128), 1))
    iota_ref[...] = iota
    valid = iota < _N
    dists_ref[...] = jnp.where(valid, jnp.inf, -jnp.inf).astype(jnp.float32)
    idx_ref[0] = 0

    def body(i, last):
        sx = ps_ref[0, last]
        sy = ps_ref[1, last]
        sz = ps_ref[2, last]
        dx = px_ref[...] - sx
        dy = py_ref[...] - sy
        dz = pz_ref[...] - sz
        d = (dx * dx + dz * dz) + dy * dy
        dn = jnp.minimum(dists_ref[...], d)
        dists_ref[...] = dn
        m = jnp.max(dn)
        nxt = jnp.min(jnp.where(dn == m, iota_ref[...], jnp.int32(0x7FFFFFFF)))
        idx_ref[i] = nxt
        return nxt

    jax.lax.fori_loop(1, _M, body, jnp.int32(0))


def _fps(pos):
    px = jnp.zeros((_NPAD,), jnp.float32).at[:_N].set(pos[:, 0]).reshape(_R, 128)
    py = jnp.zeros((_NPAD,), jnp.float32).at[:_N].set(pos[:, 1]).reshape(_R, 128)
    pz = jnp.zeros((_NPAD,), jnp.float32).at[:_N].set(pos[:, 2]).reshape(_R, 128)
    return pl.pallas_call(
        _fps_body,
        out_shape=jax.ShapeDtypeStruct((_M,), jnp.int32),
        in_specs=[
            pl.BlockSpec(memory_space=pltpu.VMEM),
            pl.BlockSpec(memory_space=pltpu.VMEM),
            pl.BlockSpec(memory_space=pltpu.VMEM),
            pl.BlockSpec(memory_space=pltpu.SMEM),
        ],
        out_specs=pl.BlockSpec(memory_space=pltpu.SMEM),
        scratch_shapes=[
            pltpu.VMEM((_R, 128), jnp.float32),
            pltpu.VMEM((_R, 128), jnp.int32),
        ],
    )(px, py, pz, pos.T)


_BN = 1024
_NBLK = 49


def _mlp_body(x_ref, rel_ref, wx_ref, wp_ref, b1_ref, w2_ref, b2_ref, o_ref):
    acc = jnp.dot(x_ref[...], wx_ref[...], preferred_element_type=jnp.float32)
    rel = rel_ref[...]
    acc += rel[:, 0:1] * wp_ref[0:1, :]
    acc += rel[:, 1:2] * wp_ref[1:2, :]
    acc += rel[:, 2:3] * wp_ref[2:3, :]
    h = jnp.maximum(acc + b1_ref[...], 0.0)
    o_ref[...] = jnp.dot(h, w2_ref[...], preferred_element_type=jnp.float32) + b2_ref[...]


def _mlp(x, rel, W1, b1, W2, b2):
    npad = _NBLK * _BN
    xp = jnp.zeros((npad, _D), jnp.float32).at[:_N].set(x)
    relp = jnp.zeros((npad, 3), jnp.float32).at[:_N].set(rel)
    h = pl.pallas_call(
        _mlp_body,
        grid=(_NBLK,),
        in_specs=[
            pl.BlockSpec((_BN, _D), lambda i: (i, 0)),
            pl.BlockSpec((_BN, 3), lambda i: (i, 0)),
            pl.BlockSpec((_D, _H), lambda i: (0, 0)),
            pl.BlockSpec((3, _H), lambda i: (0, 0)),
            pl.BlockSpec((1, _H), lambda i: (0, 0)),
            pl.BlockSpec((_H, _H), lambda i: (0, 0)),
            pl.BlockSpec((1, _H), lambda i: (0, 0)),
        ],
        out_specs=pl.BlockSpec((_BN, _H), lambda i: (i, 0)),
        out_shape=jax.ShapeDtypeStruct((npad, _H), jnp.float32),
    )(xp, relp, W1[:_D], W1[_D:], b1.reshape(1, _H), W2, b2.reshape(1, _H))
    return h[:_N]


_NW = 32
_CHUNK = 1664
_NPAD2 = _CHUNK * _NW
_MCHUNK = 512
_MPAD = _MCHUNK * _NW


def _sc_compiler_params():
    cp = pltpu.CompilerParams()
    if "needs_layout_passes" in pltpu.CompilerParams.__dataclass_fields__:
        cp = dataclasses.replace(cp, needs_layout_passes=False)
    return cp


def _gather_body(px_hbm, py_hbm, pz_hbm, bt_hbm, idx_hbm, row_hbm,
                 rx_hbm, ry_hbm, rz_hbm, px2_hbm, py2_hbm, pz2_hbm, bn_hbm,
                 idx_v, row_v, g_v, plane_v, relf_v, pn_v, bn_v):
    wid = lax.axis_index("s") * 2 + lax.axis_index("c")
    base = pl.multiple_of(wid * _CHUNK, 128)
    mbase = pl.multiple_of(wid * _MCHUNK, 128)
    pltpu.sync_copy(idx_hbm, idx_v)
    pltpu.sync_copy(row_hbm.at[pl.ds(base, _CHUNK)], row_v)

    @pl.loop(0, _CHUNK // 16)
    def _(j):
        o = pl.multiple_of(j * 16, 8)
        r16 = row_v[pl.ds(o, 16)]
        g_v[pl.ds(o, 16)] = plsc.load_gather(idx_v, [r16])

    for plane_hbm, rel_out, pn_out in ((px_hbm, rx_hbm, px2_hbm),
                                       (py_hbm, ry_hbm, py2_hbm),
                                       (pz_hbm, rz_hbm, pz2_hbm)):
        pltpu.sync_copy(plane_hbm, plane_v)

        @pl.loop(0, _CHUNK // 16)
        def _(j):
            o = pl.multiple_of(j * 16, 8)
            g16 = g_v[pl.ds(o, 16)]
            s16 = plsc.bitcast(plsc.load_gather(plane_v, [g16]), jnp.float32)
            own = plsc.bitcast(plane_v[pl.ds(pl.multiple_of(base + o, 8), 16)],
                               jnp.float32)
            relf_v[pl.ds(o, 16)] = own - s16

        pltpu.sync_copy(relf_v, rel_out.at[pl.ds(base, _CHUNK)])

        @pl.loop(0, _MCHUNK // 16)
        def _(k):
            o = pl.multiple_of(k * 16, 8)
            i16 = idx_v[pl.ds(pl.multiple_of(mbase + o, 8), 16)]
            pn_v[pl.ds(o, 16)] = plsc.bitcast(plsc.load_gather(plane_v, [i16]),
                                              jnp.float32)

        pltpu.sync_copy(pn_v, pn_out.at[pl.ds(mbase, _MCHUNK)])

    pltpu.sync_copy(bt_hbm, plane_v)

    @pl.loop(0, _MCHUNK // 16)
    def _(k):
        o = pl.multiple_of(k * 16, 8)
        i16 = idx_v[pl.ds(pl.multiple_of(mbase + o, 8), 16)]
        bn_v[pl.ds(o, 16)] = plsc.load_gather(plane_v, [i16])

    pltpu.sync_copy(bn_v, bn_hbm.at[pl.ds(mbase, _MCHUNK)])


def _sc_gather(pos, batch, idx, row):
    pos_bits = jax.lax.bitcast_convert_type(pos, jnp.int32)
    zeros_n = jnp.zeros((_NPAD2,), jnp.int32)
    px = zeros_n.at[:_N].set(pos_bits[:, 0])
    py = zeros_n.at[:_N].set(pos_bits[:, 1])
    pz = zeros_n.at[:_N].set(pos_bits[:, 2])
    bt = zeros_n.at[:_N].set(batch)
    idx_p = jnp.zeros((_MPAD,), jnp.int32).at[:_M].set(idx)
    row_p = zeros_n.at[:_N].set(row)

    kfn = pl.kernel(
        _gather_body,
        out_type=(
            jax.ShapeDtypeStruct((_NPAD2,), jnp.float32),
            jax.ShapeDtypeStruct((_NPAD2,), jnp.float32),
            jax.ShapeDtypeStruct((_NPAD2,), jnp.float32),
            jax.ShapeDtypeStruct((_MPAD,), jnp.float32),
            jax.ShapeDtypeStruct((_MPAD,), jnp.float32),
            jax.ShapeDtypeStruct((_MPAD,), jnp.float32),
            jax.ShapeDtypeStruct((_MPAD,), jnp.int32),
        ),
        mesh=plsc.VectorSubcoreMesh(core_axis_name="c", subcore_axis_name="s"),
        compiler_params=_sc_compiler_params(),
        scratch_types=[
            pltpu.VMEM((_MPAD,), jnp.int32),
            pltpu.VMEM((_CHUNK,), jnp.int32),
            pltpu.VMEM((_CHUNK,), jnp.int32),
            pltpu.VMEM((_NPAD2,), jnp.int32),
            pltpu.VMEM((_CHUNK,), jnp.float32),
            pltpu.VMEM((_MCHUNK,), jnp.float32),
            pltpu.VMEM((_MCHUNK,), jnp.int32),
        ],
    )
    rx, ry, rz, pnx, pny, pnz, bn = kfn(px, py, pz, bt, idx_p, row_p)
    rel = jnp.stack([rx[:_N], ry[:_N], rz[:_N]], axis=1)
    pos_dst = jnp.stack([pnx[:_M], pny[:_M], pnz[:_M]], axis=1)
    return rel, pos_dst, bn[:_M]


def kernel(x, pos, residue_number, batch, W1, b1, W2, b2):
    idx = _fps(pos)
    row = residue_number
    rel, pos_dst, batch_new = _sc_gather(pos, batch, idx, row)
    h = _mlp(x, rel, W1, b1, W2, b2)
    out = jax.ops.segment_max(h, row, num_segments=_M)
    out = jnp.where(jnp.isfinite(out), out, 0.0)
    return out, pos_dst, batch_new


# speedup vs baseline: 33.7528x; 1.3108x over previous
import dataclasses
import functools

import jax
import jax.numpy as jnp
from jax import lax
from jax.experimental import pallas as pl
from jax.experimental.pallas import tpu as pltpu
from jax.experimental.pallas import tpu_sc as plsc

_N = 50000
_D = 128
_M = 12500
_H = 64
_K = 49
_NPAD = _K * 1024


def _fps_body(px_ref, py_ref, pz_ref, ps_ref, idx_ref, dists_ref, iotaf_ref):
    iota = (jax.lax.broadcasted_iota(jnp.int32, (_K, 8, 128), 0) * 1024
            + jax.lax.broadcasted_iota(jnp.int32, (_K, 8, 128), 1) * 128
            + jax.lax.broadcasted_iota(jnp.int32, (_K, 8, 128), 2))
    iotaf_ref[...] = iota.astype(jnp.float32)
    valid = iota < _N
    dists_ref[...] = jnp.where(valid, jnp.inf, -jnp.inf).astype(jnp.float32)
    idx_ref[0] = 0
    inf = jnp.float32(jnp.inf)

    def body(i, last):
        sx = ps_ref[0, last]
        sy = ps_ref[1, last]
        sz = ps_ref[2, last]
        accs = [None, None, None, None]
        pays = [None, None, None, None]
        for k in range(_K):
            dx = px_ref[k] - sx
            dy = py_ref[k] - sy
            dz = pz_ref[k] - sz
            d = (dx * dx + dz * dz) + dy * dy
            dn = jnp.minimum(dists_ref[k], d)
            dists_ref[k] = dn
            a = min(k // 13, 3)
            if accs[a] is None:
                accs[a] = dn
                pays[a] = iotaf_ref[k]
            else:
                better = dn > accs[a]
                accs[a] = jnp.where(better, dn, accs[a])
                pays[a] = jnp.where(better, iotaf_ref[k], pays[a])
        b01 = accs[1] > accs[0]
        v01 = jnp.where(b01, accs[1], accs[0])
        p01 = jnp.where(b01, pays[1], pays[0])
        b23 = accs[3] > accs[2]
        v23 = jnp.where(b23, accs[3], accs[2])
        p23 = jnp.where(b23, pays[3], pays[2])
        bb = v23 > v01
        v = jnp.where(bb, v23, v01)
        p = jnp.where(bb, p23, p01)
        for sh in (4, 2, 1):
            vr = pltpu.roll(v, sh, 0)
            pr = pltpu.roll(p, sh, 0)
            better = (vr > v) | ((vr == v) & (pr < p))
            v = jnp.where(better, vr, v)
            p = jnp.where(better, pr, p)
        m = jnp.max(v, axis=(0, 1), keepdims=True)
        cand = jnp.where(v == m, p, inf)
        nxt = jnp.min(cand).astype(jnp.int32)
        idx_ref[i] = nxt
        return nxt

    jax.lax.fori_loop(1, _M, body, jnp.int32(0))


def _fps(pos):
    px = jnp.zeros((_NPAD,), jnp.float32).at[:_N].set(pos[:, 0]).reshape(_K, 8, 128)
    py = jnp.zeros((_NPAD,), jnp.float32).at[:_N].set(pos[:, 1]).reshape(_K, 8, 128)
    pz = jnp.zeros((_NPAD,), jnp.float32).at[:_N].set(pos[:, 2]).reshape(_K, 8, 128)
    return pl.pallas_call(
        _fps_body,
        out_shape=jax.ShapeDtypeStruct((_M,), jnp.int32),
        in_specs=[
            pl.BlockSpec(memory_space=pltpu.VMEM),
            pl.BlockSpec(memory_space=pltpu.VMEM),
            pl.BlockSpec(memory_space=pltpu.VMEM),
            pl.BlockSpec(memory_space=pltpu.SMEM),
        ],
        out_specs=pl.BlockSpec(memory_space=pltpu.SMEM),
        scratch_shapes=[
            pltpu.VMEM((_K, 8, 128), jnp.float32),
            pltpu.VMEM((_K, 8, 128), jnp.float32),
        ],
    )(px, py, pz, pos.T)


_BN = 1024
_NBLK = 49


def _mlp_body(x_ref, rel_ref, wx_ref, wp_ref, b1_ref, w2_ref, b2_ref, o_ref):
    acc = jnp.dot(x_ref[...], wx_ref[...], preferred_element_type=jnp.float32)
    rel = rel_ref[...]
    acc += rel[:, 0:1] * wp_ref[0:1, :]
    acc += rel[:, 1:2] * wp_ref[1:2, :]
    acc += rel[:, 2:3] * wp_ref[2:3, :]
    h = jnp.maximum(acc + b1_ref[...], 0.0)
    o_ref[...] = jnp.dot(h, w2_ref[...], preferred_element_type=jnp.float32) + b2_ref[...]


def _mlp(x, rel, W1, b1, W2, b2):
    npad = _NBLK * _BN
    xp = jnp.zeros((npad, _D), jnp.float32).at[:_N].set(x)
    relp = jnp.zeros((npad, 3), jnp.float32).at[:_N].set(rel)
    h = pl.pallas_call(
        _mlp_body,
        grid=(_NBLK,),
        in_specs=[
            pl.BlockSpec((_BN, _D), lambda i: (i, 0)),
            pl.BlockSpec((_BN, 3), lambda i: (i, 0)),
            pl.BlockSpec((_D, _H), lambda i: (0, 0)),
            pl.BlockSpec((3, _H), lambda i: (0, 0)),
            pl.BlockSpec((1, _H), lambda i: (0, 0)),
            pl.BlockSpec((_H, _H), lambda i: (0, 0)),
            pl.BlockSpec((1, _H), lambda i: (0, 0)),
        ],
        out_specs=pl.BlockSpec((_BN, _H), lambda i: (i, 0)),
        out_shape=jax.ShapeDtypeStruct((npad, _H), jnp.float32),
    )(xp, relp, W1[:_D], W1[_D:], b1.reshape(1, _H), W2, b2.reshape(1, _H))
    return h[:_N]


_NW = 32
_CHUNK = 1664
_NPAD2 = _CHUNK * _NW
_MCHUNK = 512
_MPAD = _MCHUNK * _NW


def _sc_compiler_params():
    cp = pltpu.CompilerParams()
    if "needs_layout_passes" in pltpu.CompilerParams.__dataclass_fields__:
        cp = dataclasses.replace(cp, needs_layout_passes=False)
    return cp


def _gather_body(px_hbm, py_hbm, pz_hbm, bt_hbm, idx_hbm, row_hbm,
                 rx_hbm, ry_hbm, rz_hbm, px2_hbm, py2_hbm, pz2_hbm, bn_hbm,
                 idx_v, row_v, g_v, plane_v, relf_v, pn_v, bn_v):
    wid = lax.axis_index("s") * 2 + lax.axis_index("c")
    base = pl.multiple_of(wid * _CHUNK, 128)
    mbase = pl.multiple_of(wid * _MCHUNK, 128)
    pltpu.sync_copy(idx_hbm, idx_v)
    pltpu.sync_copy(row_hbm.at[pl.ds(base, _CHUNK)], row_v)

    @pl.loop(0, _CHUNK // 16)
    def _(j):
        o = pl.multiple_of(j * 16, 8)
        r16 = row_v[pl.ds(o, 16)]
        g_v[pl.ds(o, 16)] = plsc.load_gather(idx_v, [r16])

    for plane_hbm, rel_out, pn_out in ((px_hbm, rx_hbm, px2_hbm),
                                       (py_hbm, ry_hbm, py2_hbm),
                                       (pz_hbm, rz_hbm, pz2_hbm)):
        pltpu.sync_copy(plane_hbm, plane_v)

        @pl.loop(0, _CHUNK // 16)
        def _(j):
            o = pl.multiple_of(j * 16, 8)
            g16 = g_v[pl.ds(o, 16)]
            s16 = plsc.bitcast(plsc.load_gather(plane_v, [g16]), jnp.float32)
            own = plsc.bitcast(plane_v[pl.ds(pl.multiple_of(base + o, 8), 16)],
                               jnp.float32)
            relf_v[pl.ds(o, 16)] = own - s16

        pltpu.sync_copy(relf_v, rel_out.at[pl.ds(base, _CHUNK)])

        @pl.loop(0, _MCHUNK // 16)
        def _(k):
            o = pl.multiple_of(k * 16, 8)
            i16 = idx_v[pl.ds(pl.multiple_of(mbase + o, 8), 16)]
            pn_v[pl.ds(o, 16)] = plsc.bitcast(plsc.load_gather(plane_v, [i16]),
                                              jnp.float32)

        pltpu.sync_copy(pn_v, pn_out.at[pl.ds(mbase, _MCHUNK)])

    pltpu.sync_copy(bt_hbm, plane_v)

    @pl.loop(0, _MCHUNK // 16)
    def _(k):
        o = pl.multiple_of(k * 16, 8)
        i16 = idx_v[pl.ds(pl.multiple_of(mbase + o, 8), 16)]
        bn_v[pl.ds(o, 16)] = plsc.load_gather(plane_v, [i16])

    pltpu.sync_copy(bn_v, bn_hbm.at[pl.ds(mbase, _MCHUNK)])


def _sc_gather(pos, batch, idx, row):
    pos_bits = jax.lax.bitcast_convert_type(pos, jnp.int32)
    zeros_n = jnp.zeros((_NPAD2,), jnp.int32)
    px = zeros_n.at[:_N].set(pos_bits[:, 0])
    py = zeros_n.at[:_N].set(pos_bits[:, 1])
    pz = zeros_n.at[:_N].set(pos_bits[:, 2])
    bt = zeros_n.at[:_N].set(batch)
    idx_p = jnp.zeros((_MPAD,), jnp.int32).at[:_M].set(idx)
    row_p = zeros_n.at[:_N].set(row)

    kfn = pl.kernel(
        _gather_body,
        out_type=(
            jax.ShapeDtypeStruct((_NPAD2,), jnp.float32),
            jax.ShapeDtypeStruct((_NPAD2,), jnp.float32),
            jax.ShapeDtypeStruct((_NPAD2,), jnp.float32),
            jax.ShapeDtypeStruct((_MPAD,), jnp.float32),
            jax.ShapeDtypeStruct((_MPAD,), jnp.float32),
            jax.ShapeDtypeStruct((_MPAD,), jnp.float32),
            jax.ShapeDtypeStruct((_MPAD,), jnp.int32),
        ),
        mesh=plsc.VectorSubcoreMesh(core_axis_name="c", subcore_axis_name="s"),
        compiler_params=_sc_compiler_params(),
        scratch_types=[
            pltpu.VMEM((_MPAD,), jnp.int32),
            pltpu.VMEM((_CHUNK,), jnp.int32),
            pltpu.VMEM((_CHUNK,), jnp.int32),
            pltpu.VMEM((_NPAD2,), jnp.int32),
            pltpu.VMEM((_CHUNK,), jnp.float32),
            pltpu.VMEM((_MCHUNK,), jnp.float32),
            pltpu.VMEM((_MCHUNK,), jnp.int32),
        ],
    )
    rx, ry, rz, pnx, pny, pnz, bn = kfn(px, py, pz, bt, idx_p, row_p)
    rel = jnp.stack([rx[:_N], ry[:_N], rz[:_N]], axis=1)
    pos_dst = jnp.stack([pnx[:_M], pny[:_M], pnz[:_M]], axis=1)
    return rel, pos_dst, bn[:_M]


def kernel(x, pos, residue_number, batch, W1, b1, W2, b2):
    idx = _fps(pos)
    row = residue_number
    rel, pos_dst, batch_new = _sc_gather(pos, batch, idx, row)
    h = _mlp(x, rel, W1, b1, W2, b2)
    out = jax.ops.segment_max(h, row, num_segments=_M)
    out = jnp.where(jnp.isfinite(out), out, 0.0)
    return out, pos_dst, batch_new


# speedup vs baseline: 38.0589x; 1.1276x over previous
import dataclasses
import functools

import jax
import jax.numpy as jnp
from jax import lax
from jax.experimental import pallas as pl
from jax.experimental.pallas import tpu as pltpu
from jax.experimental.pallas import tpu_sc as plsc

_N = 50000
_D = 128
_M = 12500
_H = 64
_K = 49
_NPAD = _K * 1024


def _fps_body(px_ref, py_ref, pz_ref, ps_ref, idx_ref, dists_ref, iotaf_ref):
    iota = (jax.lax.broadcasted_iota(jnp.int32, (_K, 8, 128), 0) * 1024
            + jax.lax.broadcasted_iota(jnp.int32, (_K, 8, 128), 1) * 128
            + jax.lax.broadcasted_iota(jnp.int32, (_K, 8, 128), 2))
    iotaf_ref[...] = iota.astype(jnp.float32)
    valid = iota < _N
    dists_ref[...] = jnp.where(valid, jnp.inf, -jnp.inf).astype(jnp.float32)
    idx_ref[0] = 0
    inf = jnp.float32(jnp.inf)

    def body(i, last):
        sx = ps_ref[0, last]
        sy = ps_ref[1, last]
        sz = ps_ref[2, last]
        accs = [None, None, None, None]
        pays = [None, None, None, None]
        for k in range(_K):
            dx = px_ref[k] - sx
            dy = py_ref[k] - sy
            dz = pz_ref[k] - sz
            d = (dx * dx + dz * dz) + dy * dy
            dn = jnp.minimum(dists_ref[k], d)
            dists_ref[k] = dn
            a = min(k // 13, 3)
            if accs[a] is None:
                accs[a] = dn
                pays[a] = iotaf_ref[k]
            else:
                better = dn > accs[a]
                accs[a] = jnp.where(better, dn, accs[a])
                pays[a] = jnp.where(better, iotaf_ref[k], pays[a])
        b01 = accs[1] > accs[0]
        v01 = jnp.where(b01, accs[1], accs[0])
        p01 = jnp.where(b01, pays[1], pays[0])
        b23 = accs[3] > accs[2]
        v23 = jnp.where(b23, accs[3], accs[2])
        p23 = jnp.where(b23, pays[3], pays[2])
        bb = v23 > v01
        v = jnp.where(bb, v23, v01)
        p = jnp.where(bb, p23, p01)
        for sh in (4, 2, 1):
            vr = pltpu.roll(v, sh, 0)
            pr = pltpu.roll(p, sh, 0)
            better = (vr > v) | ((vr == v) & (pr < p))
            v = jnp.where(better, vr, v)
            p = jnp.where(better, pr, p)
        m = jnp.max(v, axis=1, keepdims=True)
        cand = jnp.where(v == m, p, inf)
        nxt = jnp.min(cand).astype(jnp.int32)
        idx_ref[i] = nxt
        return nxt

    jax.lax.fori_loop(1, _M, body, jnp.int32(0))


def _fps(pos):
    px = jnp.zeros((_NPAD,), jnp.float32).at[:_N].set(pos[:, 0]).reshape(_K, 8, 128)
    py = jnp.zeros((_NPAD,), jnp.float32).at[:_N].set(pos[:, 1]).reshape(_K, 8, 128)
    pz = jnp.zeros((_NPAD,), jnp.float32).at[:_N].set(pos[:, 2]).reshape(_K, 8, 128)
    return pl.pallas_call(
        _fps_body,
        out_shape=jax.ShapeDtypeStruct((_M,), jnp.int32),
        in_specs=[
            pl.BlockSpec(memory_space=pltpu.VMEM),
            pl.BlockSpec(memory_space=pltpu.VMEM),
            pl.BlockSpec(memory_space=pltpu.VMEM),
            pl.BlockSpec(memory_space=pltpu.SMEM),
        ],
        out_specs=pl.BlockSpec(memory_space=pltpu.SMEM),
        scratch_shapes=[
            pltpu.VMEM((_K, 8, 128), jnp.float32),
            pltpu.VMEM((_K, 8, 128), jnp.float32),
        ],
    )(px, py, pz, pos.T)


_BN = 1024
_NBLK = 49


def _mlp_body(x_ref, rel_ref, wx_ref, wp_ref, b1_ref, w2_ref, b2_ref, o_ref):
    acc = jnp.dot(x_ref[...], wx_ref[...], preferred_element_type=jnp.float32)
    rel = rel_ref[...]
    acc += rel[:, 0:1] * wp_ref[0:1, :]
    acc += rel[:, 1:2] * wp_ref[1:2, :]
    acc += rel[:, 2:3] * wp_ref[2:3, :]
    h = jnp.maximum(acc + b1_ref[...], 0.0)
    o_ref[...] = jnp.dot(h, w2_ref[...], preferred_element_type=jnp.float32) + b2_ref[...]


def _mlp(x, rel, W1, b1, W2, b2):
    npad = _NBLK * _BN
    xp = jnp.zeros((npad, _D), jnp.float32).at[:_N].set(x)
    relp = jnp.zeros((npad, 3), jnp.float32).at[:_N].set(rel)
    h = pl.pallas_call(
        _mlp_body,
        grid=(_NBLK,),
        in_specs=[
            pl.BlockSpec((_BN, _D), lambda i: (i, 0)),
            pl.BlockSpec((_BN, 3), lambda i: (i, 0)),
            pl.BlockSpec((_D, _H), lambda i: (0, 0)),
            pl.BlockSpec((3, _H), lambda i: (0, 0)),
            pl.BlockSpec((1, _H), lambda i: (0, 0)),
            pl.BlockSpec((_H, _H), lambda i: (0, 0)),
            pl.BlockSpec((1, _H), lambda i: (0, 0)),
        ],
        out_specs=pl.BlockSpec((_BN, _H), lambda i: (i, 0)),
        out_shape=jax.ShapeDtypeStruct((npad, _H), jnp.float32),
    )(xp, relp, W1[:_D], W1[_D:], b1.reshape(1, _H), W2, b2.reshape(1, _H))
    return h[:_N]


_NW = 32
_CHUNK = 1664
_NPAD2 = _CHUNK * _NW
_MCHUNK = 512
_MPAD = _MCHUNK * _NW


def _sc_compiler_params():
    cp = pltpu.CompilerParams()
    if "needs_layout_passes" in pltpu.CompilerParams.__dataclass_fields__:
        cp = dataclasses.replace(cp, needs_layout_passes=False)
    return cp


def _gather_body(px_hbm, py_hbm, pz_hbm, bt_hbm, idx_hbm, row_hbm,
                 rx_hbm, ry_hbm, rz_hbm, px2_hbm, py2_hbm, pz2_hbm, bn_hbm,
                 idx_v, row_v, g_v, plane_v, relf_v, pn_v, bn_v):
    wid = lax.axis_index("s") * 2 + lax.axis_index("c")
    base = pl.multiple_of(wid * _CHUNK, 128)
    mbase = pl.multiple_of(wid * _MCHUNK, 128)
    pltpu.sync_copy(idx_hbm, idx_v)
    pltpu.sync_copy(row_hbm.at[pl.ds(base, _CHUNK)], row_v)

    @pl.loop(0, _CHUNK // 16)
    def _(j):
        o = pl.multiple_of(j * 16, 8)
        r16 = row_v[pl.ds(o, 16)]
        g_v[pl.ds(o, 16)] = plsc.load_gather(idx_v, [r16])

    for plane_hbm, rel_out, pn_out in ((px_hbm, rx_hbm, px2_hbm),
                                       (py_hbm, ry_hbm, py2_hbm),
                                       (pz_hbm, rz_hbm, pz2_hbm)):
        pltpu.sync_copy(plane_hbm, plane_v)

        @pl.loop(0, _CHUNK // 16)
        def _(j):
            o = pl.multiple_of(j * 16, 8)
            g16 = g_v[pl.ds(o, 16)]
            s16 = plsc.bitcast(plsc.load_gather(plane_v, [g16]), jnp.float32)
            own = plsc.bitcast(plane_v[pl.ds(pl.multiple_of(base + o, 8), 16)],
                               jnp.float32)
            relf_v[pl.ds(o, 16)] = own - s16

        pltpu.sync_copy(relf_v, rel_out.at[pl.ds(base, _CHUNK)])

        @pl.loop(0, _MCHUNK // 16)
        def _(k):
            o = pl.multiple_of(k * 16, 8)
            i16 = idx_v[pl.ds(pl.multiple_of(mbase + o, 8), 16)]
            pn_v[pl.ds(o, 16)] = plsc.bitcast(plsc.load_gather(plane_v, [i16]),
                                              jnp.float32)

        pltpu.sync_copy(pn_v, pn_out.at[pl.ds(mbase, _MCHUNK)])

    pltpu.sync_copy(bt_hbm, plane_v)

    @pl.loop(0, _MCHUNK // 16)
    def _(k):
        o = pl.multiple_of(k * 16, 8)
        i16 = idx_v[pl.ds(pl.multiple_of(mbase + o, 8), 16)]
        bn_v[pl.ds(o, 16)] = plsc.load_gather(plane_v, [i16])

    pltpu.sync_copy(bn_v, bn_hbm.at[pl.ds(mbase, _MCHUNK)])


def _sc_gather(pos, batch, idx, row):
    pos_bits = jax.lax.bitcast_convert_type(pos, jnp.int32)
    zeros_n = jnp.zeros((_NPAD2,), jnp.int32)
    px = zeros_n.at[:_N].set(pos_bits[:, 0])
    py = zeros_n.at[:_N].set(pos_bits[:, 1])
    pz = zeros_n.at[:_N].set(pos_bits[:, 2])
    bt = zeros_n.at[:_N].set(batch)
    idx_p = jnp.zeros((_MPAD,), jnp.int32).at[:_M].set(idx)
    row_p = zeros_n.at[:_N].set(row)

    kfn = pl.kernel(
        _gather_body,
        out_type=(
            jax.ShapeDtypeStruct((_NPAD2,), jnp.float32),
            jax.ShapeDtypeStruct((_NPAD2,), jnp.float32),
            jax.ShapeDtypeStruct((_NPAD2,), jnp.float32),
            jax.ShapeDtypeStruct((_MPAD,), jnp.float32),
            jax.ShapeDtypeStruct((_MPAD,), jnp.float32),
            jax.ShapeDtypeStruct((_MPAD,), jnp.float32),
            jax.ShapeDtypeStruct((_MPAD,), jnp.int32),
        ),
        mesh=plsc.VectorSubcoreMesh(core_axis_name="c", subcore_axis_name="s"),
        compiler_params=_sc_compiler_params(),
        scratch_types=[
            pltpu.VMEM((_MPAD,), jnp.int32),
            pltpu.VMEM((_CHUNK,), jnp.int32),
            pltpu.VMEM((_CHUNK,), jnp.int32),
            pltpu.VMEM((_NPAD2,), jnp.int32),
            pltpu.VMEM((_CHUNK,), jnp.float32),
            pltpu.VMEM((_MCHUNK,), jnp.float32),
            pltpu.VMEM((_MCHUNK,), jnp.int32),
        ],
    )
    rx, ry, rz, pnx, pny, pnz, bn = kfn(px, py, pz, bt, idx_p, row_p)
    rel = jnp.stack([rx[:_N], ry[:_N], rz[:_N]], axis=1)
    pos_dst = jnp.stack([pnx[:_M], pny[:_M], pnz[:_M]], axis=1)
    return rel, pos_dst, bn[:_M]


def kernel(x, pos, residue_number, batch, W1, b1, W2, b2):
    idx = _fps(pos)
    row = residue_number
    rel, pos_dst, batch_new = _sc_gather(pos, batch, idx, row)
    h = _mlp(x, rel, W1, b1, W2, b2)
    out = jax.ops.segment_max(h, row, num_segments=_M)
    out = jnp.where(jnp.isfinite(out), out, 0.0)
    return out, pos_dst, batch_new
